# baseline (speedup 1.0000x reference)
import functools

import jax
import jax.numpy as jnp
from jax import lax
from jax.experimental import pallas as pl
from jax.experimental.pallas import tpu as pltpu
from jax.experimental.pallas import tpu_sc as plsc

NC = 2
NS = 16
NW = NC * NS
LANES = 16

VOCAB = 1000000
EMBED = 64
BATCH = 4096
CTX = 20

RPW = BATCH // NW
CHUNK = 32
NCHUNK = RPW // CHUNK
GPC = CHUNK * CTX // 128
EV = EMBED // LANES
NBUF = 2
KT = EMBED // 8


NBLK = (VOCAB + 127) // 128
BPW = NBLK // NW
NXTRA = NBLK - BPW * NW
TAILW = VOCAB - (NBLK - 1) * 128


def _prep_body(wt_in3, wt_out3, tgt_idx, w_tail, w_lin, tgt_out,
               stage, obuf, tstage, tidx_v, tout_v, isems, osems, tsems):
  wid = lax.axis_index("s") * NC + lax.axis_index("c")
  lane = lax.iota(jnp.int32, LANES)

  i0 = []
  i1 = []
  for ev in range(EV):
    e = ev * LANES + lane
    i0.append(e >> 3)
    i1.append(e & 7)

  def transpose_block(src, dst, width):
    def tr_body(r, _):
      ci = jnp.full((LANES,), r, jnp.int32)
      for ev in range(EV):
        row = plsc.load_gather(src, [i0[ev], i1[ev], ci])
        dst[r, pl.ds(ev * LANES, LANES)] = row
      return 0
    lax.fori_loop(0, width, tr_body, 0, unroll=8)

  base = wid * BPW

  def issue_in(blk, slot):
    col = pl.multiple_of(blk * 128, 128)
    pltpu.async_copy(
        wt_in3.at[:, :, pl.ds(col, 128)],
        stage.at[slot, :, :, pl.ds(0, 128)], isems.at[slot])

  def drain_in(slot):
    pltpu.make_async_copy(
        wt_in3.at[:, :, pl.ds(0, 128)],
        stage.at[slot, :, :, pl.ds(0, 128)], isems.at[slot]).wait()

  def issue_out(blk, slot):
    row0 = pl.multiple_of(blk * 128, 128)
    pltpu.async_copy(
        obuf.at[slot], w_lin.at[pl.ds(row0, 128), :], osems.at[slot])

  def drain_out(slot):
    pltpu.make_async_copy(
        obuf.at[slot], w_lin.at[pl.ds(0, 128), :], osems.at[slot]).wait()

  for slot in range(2):
    issue_in(base + slot, slot)

  def blk_pair(t2, _):
    for slot in range(2):
      blk = base + t2 * 2 + slot

      drain_in(slot)

      @pl.when(t2 > 0)
      def _reuse(slot=slot):
        drain_out(slot)

      transpose_block(stage.at[slot], obuf.at[slot], 128)
      issue_out(blk, slot)

      @pl.when(t2 < BPW // 2 - 1)
      def _nxt(slot=slot, blk=blk):
        issue_in(blk + 2, slot)

    return 0

  lax.fori_loop(0, BPW // 2, blk_pair, 0)

  @pl.when(wid < NXTRA - 1)
  def _extra():
    blk = jnp.int32(NW * BPW) + wid
    drain_out(0)
    issue_in(blk, 0)
    drain_in(0)
    transpose_block(stage.at[0], obuf.at[0], 128)
    issue_out(blk, 0)

  @pl.when(wid == NXTRA - 1)
  def _tail():
    drain_out(0)
    pltpu.sync_copy(w_tail, obuf.at[0, pl.ds(0, TAILW)])
    pltpu.sync_copy(obuf.at[0, pl.ds(0, TAILW)],
                    w_lin.at[pl.ds((NBLK - 1) * 128, TAILW), :])

  pltpu.sync_copy(tgt_idx.at[wid], tidx_v.at[pl.ds(0, RPW)])

  def scal(v, l):
    return jnp.sum(jnp.where(lane == l, v, jnp.int32(0)))

  def t_issue(col, slot):
    col = pl.multiple_of(col, 128)
    pltpu.async_copy(
        wt_out3.at[:, :, pl.ds(col, 128)],
        tstage.at[slot, :, :, pl.ds(0, 128)], tsems.at[slot])

  def t_drain(slot):
    pltpu.make_async_copy(
        wt_out3.at[:, :, pl.ds(0, 128)],
        tstage.at[slot, :, :, pl.ds(0, 128)], tsems.at[slot]).wait()

  idvec0 = tidx_v[pl.ds(0, LANES)]
  blk0 = (idvec0 >> 7) * 128
  for slot in range(NBUF):
    t_issue(scal(blk0, slot), slot)

  def grp_body(g, _):
    idcur = tidx_v[pl.ds(g * LANES, LANES)]
    civ = idcur & 127
    blkcur = (idcur >> 7) * 128
    idnext = tidx_v[pl.ds(g * LANES + LANES, LANES)]
    blknext = (idnext >> 7) * 128
    for j in range(LANES):
      r = g * LANES + j
      slot = j % NBUF
      t_drain(slot)
      ci = jnp.full((LANES,), scal(civ, j), jnp.int32)
      for ev in range(EV):
        row = plsc.load_gather(tstage.at[slot], [i0[ev], i1[ev], ci])
        tout_v[r, pl.ds(ev * LANES, LANES)] = row

      if j < LANES - NBUF:
        t_issue(scal(blkcur, j + NBUF), slot)
      else:
        @pl.when(g < RPW // LANES - 1)
        def _nt(slot=slot, blknext=blknext, j=j):
          t_issue(scal(blknext, j + NBUF - LANES), slot)

    return 0

  lax.fori_loop(0, RPW // LANES, grp_body, 0)
  pltpu.sync_copy(tout_v, tgt_out.at[wid])

  drain_out(1)

  @pl.when(wid != NXTRA - 1)
  def _fin():
    drain_out(0)


def _cbow_body(w_in, ctx_idx, tgt_hbm, out, idx_v, ctx_rows, tgt_rows,
               scores_v, sem):
  wid = lax.axis_index("s") * NC + lax.axis_index("c")

  pltpu.sync_copy(tgt_hbm.at[wid], tgt_rows)

  inv_ctx = jnp.float32(1.0 / CTX)
  lane_iota = lax.iota(jnp.int32, LANES)

  for chunk in range(NCHUNK):
    pltpu.sync_copy(ctx_idx.at[wid, chunk], idx_v)
    copies = []
    for j in range(GPC):
      copies.append(
          pltpu.async_copy(w_in.at[idx_v.at[j]],
                           ctx_rows.at[pl.ds(j * 128, 128)], sem))
    for c in copies:
      c.wait()

    def row_body(r, svec, chunk=chunk):
      base = r * CTX
      trow = chunk * CHUNK + r
      prod = None
      for e in range(EV):
        acc = ctx_rows[base, pl.ds(e * LANES, LANES)]
        for c in range(1, CTX):
          acc = acc + ctx_rows[base + c, pl.ds(e * LANES, LANES)]
        term = acc * tgt_rows[trow, pl.ds(e * LANES, LANES)]
        prod = term if prod is None else prod + term
      s = jnp.sum(prod) * inv_ctx
      svec = jnp.where(lane_iota == (r & (LANES - 1)), s, svec)

      @pl.when((r & (LANES - 1)) == LANES - 1)
      def _store(svec=svec, r=r):
        scores_v[pl.ds(chunk * CHUNK + (r & ~(LANES - 1)), LANES)] = svec

      return svec

    lax.fori_loop(0, CHUNK, row_body, jnp.zeros((LANES,), jnp.float32))

  pltpu.sync_copy(scores_v, out.at[wid])


@jax.jit
def _cbow(ctx_idx, tgt_idx, w_in, w_out):
  mesh = plsc.VectorSubcoreMesh(core_axis_name="c", subcore_axis_name="s")

  wt_in3 = w_in.T.reshape(KT, 8, VOCAB)
  wt_out3 = w_out.T.reshape(KT, 8, VOCAB)
  prep_fn = pl.kernel(
      _prep_body,
      out_type=(
          jax.ShapeDtypeStruct((VOCAB, EMBED), jnp.float32),
          jax.ShapeDtypeStruct((NW, RPW, EMBED), jnp.float32),
      ),
      mesh=mesh,
      compiler_params=pltpu.CompilerParams(needs_layout_passes=False),
      scratch_types=[
          pltpu.VMEM((2, KT, 8, 129), jnp.float32),
          pltpu.VMEM((2, 128, EMBED), jnp.float32),
          pltpu.VMEM((NBUF, KT, 8, 129), jnp.float32),
          pltpu.VMEM((RPW + LANES,), jnp.int32),
          pltpu.VMEM((RPW, EMBED), jnp.float32),
          pltpu.SemaphoreType.DMA((2,)),
          pltpu.SemaphoreType.DMA((2,)),
          pltpu.SemaphoreType.DMA((NBUF,)),
      ],
  )
  w_lin, tgt_rows = prep_fn(wt_in3, wt_out3, tgt_idx,
                            w_in[VOCAB - TAILW:])

  cbow_fn = pl.kernel(
      _cbow_body,
      out_type=jax.ShapeDtypeStruct((NW, RPW), jnp.float32),
      mesh=mesh,
      compiler_params=pltpu.CompilerParams(
          needs_layout_passes=False, use_tc_tiling_on_sc=False),
      scratch_types=[
          pltpu.VMEM((GPC, 128), jnp.int32),
          pltpu.VMEM((CHUNK * CTX, EMBED), jnp.float32),
          pltpu.VMEM((RPW, EMBED), jnp.float32),
          pltpu.VMEM((RPW,), jnp.float32),
          pltpu.SemaphoreType.DMA,
      ],
  )
  return cbow_fn(w_lin, ctx_idx, tgt_rows)


def kernel(context_ids, target_ids, W_in, W_out):
  ctx_idx = context_ids.astype(jnp.int32).reshape(NW, NCHUNK, GPC, 128)
  tgt_idx = target_ids.astype(jnp.int32).reshape(NW, RPW)
  out = _cbow(ctx_idx, tgt_idx, W_in, W_out)
  return out.reshape(BATCH)


# speedup vs baseline: 3.0377x; 3.0377x over previous
import functools

import jax
import jax.numpy as jnp
from jax import lax
from jax.experimental import pallas as pl
from jax.experimental.pallas import tpu as pltpu
from jax.experimental.pallas import tpu_sc as plsc

NC = 2
NS = 16
NW = NC * NS
LANES = 16

VOCAB = 1000000
EMBED = 64
BATCH = 4096
CTX = 20

RPW = BATCH // NW
CHUNK = 32
NCHUNK = RPW // CHUNK
GPC = CHUNK * CTX // 128
EV = EMBED // LANES
NBUF = 8
KT = EMBED // 8


def _tgt_body(wt3, tgt_idx, out, tidx_v, stage, out_v, sems):
  wid = lax.axis_index("s") * NC + lax.axis_index("c")
  pltpu.sync_copy(tgt_idx.at[wid], tidx_v.at[pl.ds(0, RPW)])

  lane = lax.iota(jnp.int32, LANES)
  i0 = []
  i1 = []
  for ev in range(EV):
    e = ev * LANES + lane
    i0.append(e >> 3)
    i1.append(e & 7)

  def scal(v, l):
    return jnp.sum(jnp.where(lane == l, v, jnp.int32(0)))

  def issue(col, slot):
    col = pl.multiple_of(col, 128)
    pltpu.async_copy(
        wt3.at[:, :, pl.ds(col, 128)], stage.at[slot], sems.at[slot])

  def drain(slot):
    pltpu.make_async_copy(
        wt3.at[:, :, pl.ds(0, 128)], stage.at[slot], sems.at[slot]).wait()

  idvec0 = tidx_v[pl.ds(0, LANES)]
  blk0 = (idvec0 >> 7) * 128
  for slot in range(NBUF):
    issue(scal(blk0, slot), slot)

  def phase_body(ph, _):
    idcur = tidx_v[pl.ds(ph * NBUF, LANES)]
    civ = idcur & 127
    idnext = tidx_v[pl.ds(ph * NBUF + NBUF, LANES)]
    blknext = (idnext >> 7) * 128
    for slot in range(NBUF):
      r = ph * NBUF + slot
      drain(slot)
      ci = jnp.full((LANES,), scal(civ, slot), jnp.int32)
      for ev in range(EV):
        row = plsc.load_gather(stage.at[slot], [i0[ev], i1[ev], ci])
        out_v[r, pl.ds(ev * LANES, LANES)] = row

      @pl.when(ph < RPW // NBUF - 1)
      def _next(slot=slot, blknext=blknext):
        issue(scal(blknext, slot), slot)

    return 0

  lax.fori_loop(0, RPW // NBUF, phase_body, 0)
  pltpu.sync_copy(out_v, out.at[wid])


def _cbow_body(w_in, ctx_idx, tgt_hbm, out, idx_v, ctx_rows, tgt_rows,
               scores_v, sem):
  wid = lax.axis_index("s") * NC + lax.axis_index("c")

  pltpu.sync_copy(tgt_hbm.at[wid], tgt_rows)

  inv_ctx = jnp.float32(1.0 / CTX)
  lane_iota = lax.iota(jnp.int32, LANES)

  for chunk in range(NCHUNK):
    pltpu.sync_copy(ctx_idx.at[wid, chunk], idx_v)
    copies = []
    for j in range(GPC):
      copies.append(
          pltpu.async_copy(w_in.at[idx_v.at[j]],
                           ctx_rows.at[pl.ds(j * 128, 128)], sem))
    for c in copies:
      c.wait()

    def row_body(r, svec, chunk=chunk):
      base = r * CTX
      trow = chunk * CHUNK + r
      prod = None
      for e in range(EV):
        acc = ctx_rows[base, pl.ds(e * LANES, LANES)]
        for c in range(1, CTX):
          acc = acc + ctx_rows[base + c, pl.ds(e * LANES, LANES)]
        term = acc * tgt_rows[trow, pl.ds(e * LANES, LANES)]
        prod = term if prod is None else prod + term
      s = jnp.sum(prod) * inv_ctx
      svec = jnp.where(lane_iota == (r & (LANES - 1)), s, svec)

      @pl.when((r & (LANES - 1)) == LANES - 1)
      def _store(svec=svec, r=r):
        scores_v[pl.ds(chunk * CHUNK + (r & ~(LANES - 1)), LANES)] = svec

      return svec

    lax.fori_loop(0, CHUNK, row_body, jnp.zeros((LANES,), jnp.float32))

  pltpu.sync_copy(scores_v, out.at[wid])


@jax.jit
def _cbow(ctx_idx, tgt_idx, w_in, w_out):
  mesh = plsc.VectorSubcoreMesh(core_axis_name="c", subcore_axis_name="s")

  wt3 = w_out.T.reshape(KT, 8, VOCAB)
  tgt_fn = pl.kernel(
      _tgt_body,
      out_type=jax.ShapeDtypeStruct((NW, RPW, EMBED), jnp.float32),
      mesh=mesh,
      compiler_params=pltpu.CompilerParams(needs_layout_passes=False),
      scratch_types=[
          pltpu.VMEM((RPW + LANES,), jnp.int32),
          pltpu.VMEM((NBUF, KT, 8, 128), jnp.float32),
          pltpu.VMEM((RPW, EMBED), jnp.float32),
          pltpu.SemaphoreType.DMA((NBUF,)),
      ],
  )
  tgt_rows = tgt_fn(wt3, tgt_idx)

  cbow_fn = pl.kernel(
      _cbow_body,
      out_type=jax.ShapeDtypeStruct((NW, RPW), jnp.float32),
      mesh=mesh,
      compiler_params=pltpu.CompilerParams(
          needs_layout_passes=False, use_tc_tiling_on_sc=False),
      scratch_types=[
          pltpu.VMEM((GPC, 128), jnp.int32),
          pltpu.VMEM((CHUNK * CTX, EMBED), jnp.float32),
          pltpu.VMEM((RPW, EMBED), jnp.float32),
          pltpu.VMEM((RPW,), jnp.float32),
          pltpu.SemaphoreType.DMA,
      ],
  )
  return cbow_fn(w_in, ctx_idx, tgt_rows)


def kernel(context_ids, target_ids, W_in, W_out):
  ctx_idx = context_ids.astype(jnp.int32).reshape(NW, NCHUNK, GPC, 128)
  tgt_idx = target_ids.astype(jnp.int32).reshape(NW, RPW)
  out = _cbow(ctx_idx, tgt_idx, W_in, W_out)
  return out.reshape(BATCH)
